# baseline (speedup 1.0000x reference)
import functools

import jax
import jax.numpy as jnp
from jax import lax
from jax.experimental import pallas as pl
from jax.experimental.pallas import tpu as pltpu
from jax.experimental.pallas import tpu_sc as plsc

N = 10000
E = 320000
D_IN = 128
D_HID = 64
NG = 128

NC = 2
NS = 16
L = 16
NW = NC * NS
EPW = E // NW
CH = 80
NCHUNK = EPW // CH
NPAIR = (NCHUNK - 3) // 2
NP = 10240
ZPT = NP // NS
ZB = 32
DEGW = 16


def _sc_mesh():
    return plsc.VectorSubcoreMesh(
        core_axis_name="c", subcore_axis_name="s",
        num_cores=NC, num_subcores=NS)


def _zero_vmem(ref, rows, d):
    def body(i, _):
        for k in range(d // L):
            ref[i, pl.ds(k * L, L)] = jnp.zeros((L,), jnp.float32)
        return 0
    lax.fori_loop(0, rows, body, 0)


@functools.partial(
    pl.kernel,
    out_type=jax.ShapeDtypeStruct((NC * NP, DEGW), jnp.float32),
    mesh=_sc_mesh(),
    scratch_types=[
        pltpu.VMEM((NCHUNK, CH), jnp.int32),
        pltpu.VMEM((CH, DEGW), jnp.float32),
        pltpu.VMEM((ZB, DEGW), jnp.float32),
        pltpu.VMEM_SHARED((NP, DEGW), jnp.float32),
        pltpu.SemaphoreType.DMA,
        pltpu.SemaphoreType.DMA,
    ],
    name="gcn_deg",
    compiler_params=pltpu.CompilerParams(use_tc_tiling_on_sc=False),
)
def _deg_kernel(dst_hbm, out_hbm, dstall, ones_v, zero_v, acc_sh, semi, sems):
    c = lax.axis_index("c")
    s = lax.axis_index("s")
    wid = s * NC + c

    pltpu.async_copy(dst_hbm.at[pl.ds(wid * NCHUNK, NCHUNK)], dstall, semi)

    def fill_ones(i, _):
        ones_v[i, :] = jnp.ones((L,), jnp.float32)
        return 0
    lax.fori_loop(0, CH, fill_ones, 0)
    _zero_vmem(zero_v, ZB, DEGW)

    def zcp(i, _):
        pltpu.sync_copy(zero_v,
                        acc_sh.at[pl.ds(pl.multiple_of(s * ZPT + i * ZB, 8), ZB)])
        return 0
    lax.fori_loop(0, ZPT // ZB, zcp, 0)
    pltpu.make_async_copy(dst_hbm.at[pl.ds(0, NCHUNK)], dstall, semi).wait()
    plsc.subcore_barrier()

    def fire(j, _):
        pltpu.async_copy(ones_v, acc_sh.at[dstall.at[j]], sems, add=True)
        return 0
    lax.fori_loop(0, NCHUNK, fire, 0)

    def drain(j, _):
        pltpu.make_async_copy(ones_v, acc_sh.at[dstall.at[j]], sems).wait()
        return 0
    lax.fori_loop(0, NCHUNK, drain, 0)

    plsc.subcore_barrier()
    row0 = pl.multiple_of(s * ZPT, 8)
    pltpu.sync_copy(acc_sh.at[pl.ds(row0, ZPT)],
                    out_hbm.at[pl.ds(pl.multiple_of(c * NP + s * ZPT, 8), ZPT)])


def _make_prop(d, nbuf):
    niter = NCHUNK // nbuf
    rem = NCHUNK % nbuf

    @functools.partial(
        pl.kernel,
        out_type=jax.ShapeDtypeStruct((NC * NP, d), jnp.float32),
        mesh=_sc_mesh(),
        scratch_types=(
            [pltpu.VMEM((NCHUNK, CH), jnp.int32),
             pltpu.VMEM((NCHUNK, CH), jnp.int32)]
            + [pltpu.VMEM((CH, d), jnp.float32)] * nbuf
            + [pltpu.VMEM((ZB, d), jnp.float32),
               pltpu.VMEM_SHARED((NP, d), jnp.float32),
               pltpu.SemaphoreType.DMA]
            + [pltpu.SemaphoreType.DMA] * nbuf
            + [pltpu.SemaphoreType.DMA] * nbuf
        ),
        name=f"gcn_prop_{d}",
        compiler_params=pltpu.CompilerParams(use_tc_tiling_on_sc=False),
    )
    def prop(src_hbm, dst_hbm, g_hbm, out_hbm, srcall, dstall, *rest):
        rowsb = rest[:nbuf]
        zero_v = rest[nbuf]
        acc_sh = rest[nbuf + 1]
        semi = rest[nbuf + 2]
        semg = rest[nbuf + 3:nbuf + 3 + nbuf]
        sems = rest[nbuf + 3 + nbuf:]

        c = lax.axis_index("c")
        s = lax.axis_index("s")
        wid = s * NC + c

        pltpu.async_copy(src_hbm.at[pl.ds(wid * NCHUNK, NCHUNK)], srcall, semi)
        pltpu.async_copy(dst_hbm.at[pl.ds(wid * NCHUNK, NCHUNK)], dstall, semi)

        _zero_vmem(zero_v, ZB, d)

        def zcp(i, _):
            pltpu.sync_copy(
                zero_v,
                acc_sh.at[pl.ds(pl.multiple_of(s * ZPT + i * ZB, 8), ZB)])
            return 0
        lax.fori_loop(0, ZPT // ZB, zcp, 0)

        pltpu.make_async_copy(src_hbm.at[pl.ds(0, NCHUNK)], srcall, semi).wait()
        pltpu.make_async_copy(dst_hbm.at[pl.ds(0, NCHUNK)], dstall, semi).wait()

        for b in range(nbuf):
            pltpu.async_copy(g_hbm.at[srcall.at[b]], rowsb[b], semg[b])
        plsc.subcore_barrier()

        def step(j, b):
            pltpu.make_async_copy(g_hbm.at[srcall.at[j]], rowsb[b],
                                  semg[b]).wait()
            pltpu.async_copy(rowsb[b], acc_sh.at[dstall.at[j]], sems[b],
                             add=True)
            pltpu.make_async_copy(rowsb[b], acc_sh.at[dstall.at[j]],
                                  sems[b]).wait()

            @pl.when(j + nbuf < NCHUNK)
            def _():
                pltpu.async_copy(g_hbm.at[srcall.at[j + nbuf]], rowsb[b],
                                 semg[b])

        def body(t, _):
            for b in range(nbuf):
                step(t * nbuf + b, b)
            return 0
        lax.fori_loop(0, niter, body, 0)
        for r in range(rem):
            step(niter * nbuf + r, r)

        plsc.subcore_barrier()
        pltpu.sync_copy(acc_sh.at[pl.ds(pl.multiple_of(s * ZPT, 8), ZPT)],
                        out_hbm.at[pl.ds(pl.multiple_of(c * NP + s * ZPT, 8),
                                         ZPT)])

    return prop


_prop128 = _make_prop(D_IN, 2)
_prop64 = _make_prop(D_HID, 4)


BR = NP // 8


def _row_spec(w):
    return pl.BlockSpec((BR, w), lambda i: (i, 0))


def _hi_spec(w):
    return pl.BlockSpec((BR, w), lambda i: (i + 8, 0))


def _full_spec():
    return pl.BlockSpec(index_map=lambda i: (0, 0))


def _k1_body(x_ref, w1_ref, d0_ref, d1_ref, g1_ref, dis_ref):
    deg = d0_ref[:, pl.ds(0, 1)] + d1_ref[:, pl.ds(0, 1)] + 1.0
    dis = lax.rsqrt(deg)
    dis_ref[...] = dis
    t1 = jnp.dot(x_ref[...], w1_ref[...], preferred_element_type=jnp.float32)
    g1_ref[...] = t1 * dis


_k1 = pl.pallas_call(
    _k1_body,
    grid=(8,),
    in_specs=[_row_spec(D_IN), _full_spec(), _row_spec(DEGW), _hi_spec(DEGW)],
    out_specs=(_row_spec(D_IN), _row_spec(1)),
    out_shape=(jax.ShapeDtypeStruct((NP, D_IN), jnp.float32),
               jax.ShapeDtypeStruct((NP, 1), jnp.float32)),
)


def _k2_body(s1a_ref, s1b_ref, g1_ref, dis_ref, b1_ref, w2_ref, g2_ref):
    dis = dis_ref[...]
    agg = s1a_ref[...] + s1b_ref[...] + g1_ref[...]
    h1 = jnp.maximum(dis * agg + b1_ref[...], 0.0)
    g2_ref[...] = jnp.dot(h1, w2_ref[...],
                          preferred_element_type=jnp.float32) * dis


_k2 = pl.pallas_call(
    _k2_body,
    grid=(8,),
    in_specs=[_row_spec(D_IN), _hi_spec(D_IN), _row_spec(D_IN), _row_spec(1),
              _full_spec(), _full_spec()],
    out_specs=_row_spec(D_HID),
    out_shape=jax.ShapeDtypeStruct((NP, D_HID), jnp.float32),
)


def _k3_body(s2a_ref, s2b_ref, g2_ref, dis_ref, b2_ref, g3_ref):
    dis = dis_ref[...]
    agg = s2a_ref[...] + s2b_ref[...] + g2_ref[...]
    h2 = jnp.maximum(dis * agg + b2_ref[...], 0.0)
    g3_ref[...] = h2 * dis


_k3 = pl.pallas_call(
    _k3_body,
    grid=(8,),
    in_specs=[_row_spec(D_HID), _hi_spec(D_HID), _row_spec(D_HID),
              _row_spec(1), _full_spec()],
    out_specs=_row_spec(D_HID),
    out_shape=jax.ShapeDtypeStruct((NP, D_HID), jnp.float32),
)


def _k4_body(s3a_ref, s3b_ref, g3_ref, dis_ref, w3_ref, b3_ref, batch_ref,
             out_ref, sums_sc, cnt_sc):
    i = pl.program_id(0)
    dis = dis_ref[...]
    p3 = dis * (s3a_ref[...] + s3b_ref[...] + g3_ref[...])
    h3 = jnp.maximum(jnp.dot(p3, w3_ref[...],
                             preferred_element_type=jnp.float32)
                     + b3_ref[...], 0.0)
    gids = lax.broadcasted_iota(jnp.int32, (NG, BR), 0)
    onehot_t = (gids == batch_ref[...]).astype(jnp.float32)
    bsums = jnp.dot(onehot_t, h3, preferred_element_type=jnp.float32)
    bcnt = jnp.sum(onehot_t, axis=1, keepdims=True)

    @pl.when(i == 0)
    def _():
        sums_sc[...] = bsums
        cnt_sc[...] = bcnt

    @pl.when(i > 0)
    def _():
        sums_sc[...] += bsums
        cnt_sc[...] += bcnt

    @pl.when(i == pl.num_programs(0) - 1)
    def _():
        out_ref[...] = sums_sc[...] / jnp.maximum(cnt_sc[...], 1.0)


_k4 = pl.pallas_call(
    _k4_body,
    grid=(8,),
    in_specs=[_row_spec(D_HID), _hi_spec(D_HID), _row_spec(D_HID),
              _row_spec(1), _full_spec(), _full_spec(),
              pl.BlockSpec((1, BR), lambda i: (0, i))],
    out_specs=pl.BlockSpec((NG, D_IN), lambda i: (0, 0)),
    out_shape=jax.ShapeDtypeStruct((NG, D_IN), jnp.float32),
    scratch_shapes=[pltpu.VMEM((NG, D_IN), jnp.float32),
                    pltpu.VMEM((NG, 1), jnp.float32)],
)


def kernel(x, edge_index, batch, W1, b1, W2, b2, W3, b3):
    src = edge_index[0].astype(jnp.int32).reshape(NW * NCHUNK, CH)
    dst = edge_index[1].astype(jnp.int32).reshape(NW * NCHUNK, CH)
    xp = jnp.concatenate(
        [x, jnp.zeros((NP - N, D_IN), jnp.float32)], axis=0)
    batch_p = jnp.concatenate(
        [batch.astype(jnp.int32), jnp.full((NP - N,), NG, jnp.int32)]
    ).reshape(1, NP)
    b1r = b1.reshape(1, -1)
    b2r = b2.reshape(1, -1)
    b3r = b3.reshape(1, -1)

    degp = _deg_kernel(dst)
    g1, dis = _k1(xp, W1, degp, degp)

    s1 = _prop128(src, dst, g1)
    g2 = _k2(s1, s1, g1, dis, b1r, W2)

    s2 = _prop64(src, dst, g2)
    g3 = _k3(s2, s2, g2, dis, b2r)

    s3 = _prop64(src, dst, g3)
    out = _k4(s3, s3, g3, dis, W3, b3r, batch_p)
    return out


# speedup vs baseline: 1.0801x; 1.0801x over previous
import functools

import jax
import jax.numpy as jnp
from jax import lax
from jax.experimental import pallas as pl
from jax.experimental.pallas import tpu as pltpu
from jax.experimental.pallas import tpu_sc as plsc

N = 10000
E = 320000
D_IN = 128
D_HID = 64
NG = 128

NC = 2
NS = 16
L = 16
NW = NC * NS
EPW = E // NW
CH = 80
NCHUNK = EPW // CH
NPAIR = (NCHUNK - 3) // 2
NP = 10240
ZPT = NP // NS
ZB = 32
DEGW = 16


def _sc_mesh():
    return plsc.VectorSubcoreMesh(
        core_axis_name="c", subcore_axis_name="s",
        num_cores=NC, num_subcores=NS)


def _zero_vmem(ref, rows, d):
    def body(i, _):
        for k in range(d // L):
            ref[i, pl.ds(k * L, L)] = jnp.zeros((L,), jnp.float32)
        return 0
    lax.fori_loop(0, rows, body, 0)


@functools.partial(
    pl.kernel,
    out_type=jax.ShapeDtypeStruct((NC * NP, DEGW), jnp.float32),
    mesh=_sc_mesh(),
    scratch_types=[
        pltpu.VMEM((NCHUNK, CH), jnp.int32),
        pltpu.VMEM((CH, DEGW), jnp.float32),
        pltpu.VMEM((ZB, DEGW), jnp.float32),
        pltpu.VMEM_SHARED((NP, DEGW), jnp.float32),
        pltpu.SemaphoreType.DMA,
        pltpu.SemaphoreType.DMA,
    ],
    name="gcn_deg",
    compiler_params=pltpu.CompilerParams(use_tc_tiling_on_sc=False),
)
def _deg_kernel(dst_hbm, out_hbm, dstall, ones_v, zero_v, acc_sh, semi, sems):
    c = lax.axis_index("c")
    s = lax.axis_index("s")
    wid = s * NC + c

    pltpu.async_copy(dst_hbm.at[pl.ds(wid * NCHUNK, NCHUNK)], dstall, semi)

    def fill_ones(i, _):
        ones_v[i, :] = jnp.ones((L,), jnp.float32)
        return 0
    lax.fori_loop(0, CH, fill_ones, 0)
    _zero_vmem(zero_v, ZB, DEGW)

    def zcp(i, _):
        pltpu.sync_copy(zero_v,
                        acc_sh.at[pl.ds(pl.multiple_of(s * ZPT + i * ZB, 8), ZB)])
        return 0
    lax.fori_loop(0, ZPT // ZB, zcp, 0)
    pltpu.make_async_copy(dst_hbm.at[pl.ds(0, NCHUNK)], dstall, semi).wait()
    plsc.subcore_barrier()

    def fire(j, _):
        pltpu.async_copy(ones_v, acc_sh.at[dstall.at[j]], sems, add=True)
        return 0
    lax.fori_loop(0, NCHUNK, fire, 0)

    def drain(j, _):
        pltpu.make_async_copy(ones_v, acc_sh.at[dstall.at[j]], sems).wait()
        return 0
    lax.fori_loop(0, NCHUNK, drain, 0)

    plsc.subcore_barrier()
    row0 = pl.multiple_of(s * ZPT, 8)
    pltpu.sync_copy(acc_sh.at[pl.ds(row0, ZPT)],
                    out_hbm.at[pl.ds(pl.multiple_of(c * NP + s * ZPT, 8), ZPT)])


def _make_prop(d, nbuf, phase_lens):
    ph = max(phase_lens)

    @functools.partial(
        pl.kernel,
        out_type=jax.ShapeDtypeStruct((NC * NP, d), jnp.float32),
        mesh=_sc_mesh(),
        scratch_types=(
            [pltpu.VMEM((ph, CH), jnp.int32),
             pltpu.VMEM((ph, CH), jnp.int32)]
            + [pltpu.VMEM((CH, d), jnp.float32)] * nbuf
            + [pltpu.VMEM((ZB, d), jnp.float32),
               pltpu.VMEM_SHARED((NP, d), jnp.float32),
               pltpu.SemaphoreType.DMA]
            + [pltpu.SemaphoreType.DMA] * nbuf
            + [pltpu.SemaphoreType.DMA] * nbuf
        ),
        name=f"gcn_prop_{d}",
        compiler_params=pltpu.CompilerParams(use_tc_tiling_on_sc=False),
    )
    def prop(src_hbm, dst_hbm, g_hbm, out_hbm, srcph, dstph, *rest):
        rowsb = rest[:nbuf]
        zero_v = rest[nbuf]
        acc_sh = rest[nbuf + 1]
        semi = rest[nbuf + 2]
        semg = rest[nbuf + 3:nbuf + 3 + nbuf]
        sems = rest[nbuf + 3 + nbuf:]

        c = lax.axis_index("c")
        s = lax.axis_index("s")
        wid = s * NC + c

        def issue_idx(row0, ln):
            base = wid * NCHUNK + row0
            pltpu.async_copy(src_hbm.at[pl.ds(base, ln)],
                             srcph.at[pl.ds(0, ln)], semi)
            pltpu.async_copy(dst_hbm.at[pl.ds(base, ln)],
                             dstph.at[pl.ds(0, ln)], semi)

        def wait_idx(ln):
            pltpu.make_async_copy(src_hbm.at[pl.ds(0, ln)],
                                  srcph.at[pl.ds(0, ln)], semi).wait()
            pltpu.make_async_copy(dst_hbm.at[pl.ds(0, ln)],
                                  dstph.at[pl.ds(0, ln)], semi).wait()

        def step(jl, b, ln):
            pltpu.make_async_copy(g_hbm.at[srcph.at[jl]], rowsb[b],
                                  semg[b]).wait()
            pltpu.async_copy(rowsb[b], acc_sh.at[dstph.at[jl]], sems[b],
                             add=True)
            pltpu.make_async_copy(rowsb[b], acc_sh.at[dstph.at[jl]],
                                  sems[b]).wait()

            @pl.when(jl + nbuf < ln)
            def _():
                pltpu.async_copy(g_hbm.at[srcph.at[jl + nbuf]], rowsb[b],
                                 semg[b])

        issue_idx(0, phase_lens[0])
        _zero_vmem(zero_v, ZB, d)

        def zcp(i, _):
            pltpu.sync_copy(
                zero_v,
                acc_sh.at[pl.ds(pl.multiple_of(s * ZPT + i * ZB, 8), ZB)])
            return 0
        lax.fori_loop(0, ZPT // ZB, zcp, 0)
        wait_idx(phase_lens[0])
        for b in range(min(nbuf, phase_lens[0])):
            pltpu.async_copy(g_hbm.at[srcph.at[b]], rowsb[b], semg[b])
        plsc.subcore_barrier()

        row0 = 0
        for pi, ln in enumerate(phase_lens):
            if pi > 0:
                issue_idx(row0, ln)
                wait_idx(ln)
                for b in range(min(nbuf, ln)):
                    pltpu.async_copy(g_hbm.at[srcph.at[b]], rowsb[b], semg[b])
            niter = ln // nbuf
            rem = ln % nbuf

            def body(t, _):
                for b in range(nbuf):
                    step(t * nbuf + b, b, ln)
                return 0
            lax.fori_loop(0, niter, body, 0)
            for r in range(rem):
                step(niter * nbuf + r, r, ln)
            row0 += ln

        plsc.subcore_barrier()
        pltpu.sync_copy(acc_sh.at[pl.ds(pl.multiple_of(s * ZPT, 8), ZPT)],
                        out_hbm.at[pl.ds(pl.multiple_of(c * NP + s * ZPT, 8),
                                         ZPT)])

    return prop


_prop128 = _make_prop(D_IN, 3, [63, 62])
_prop64 = _make_prop(D_HID, 4, [NCHUNK])


def _k1_body(x_ref, w1_ref, degp_ref, g1_ref, dis_ref):
    deg = (degp_ref[pl.ds(0, N), pl.ds(0, 1)]
           + degp_ref[pl.ds(NP, N), pl.ds(0, 1)] + 1.0)
    dis = lax.rsqrt(deg)
    dis_ref[...] = dis
    t1 = jnp.dot(x_ref[...], w1_ref[...], preferred_element_type=jnp.float32)
    g1_ref[...] = t1 * dis


_k1 = pl.pallas_call(
    _k1_body,
    out_shape=(jax.ShapeDtypeStruct((N, D_IN), jnp.float32),
               jax.ShapeDtypeStruct((N, 1), jnp.float32)),
)


def _k2_body(s1_ref, g1_ref, dis_ref, b1_ref, w2_ref, g2_ref):
    dis = dis_ref[...]
    agg = (s1_ref[pl.ds(0, N), :] + s1_ref[pl.ds(NP, N), :] + g1_ref[...])
    h1 = jnp.maximum(dis * agg + b1_ref[...], 0.0)
    g2_ref[...] = jnp.dot(h1, w2_ref[...],
                          preferred_element_type=jnp.float32) * dis


_k2 = pl.pallas_call(
    _k2_body,
    out_shape=jax.ShapeDtypeStruct((N, D_HID), jnp.float32),
)


def _k3_body(s2_ref, g2_ref, dis_ref, b2_ref, g3_ref):
    dis = dis_ref[...]
    agg = (s2_ref[pl.ds(0, N), :] + s2_ref[pl.ds(NP, N), :] + g2_ref[...])
    h2 = jnp.maximum(dis * agg + b2_ref[...], 0.0)
    g3_ref[...] = h2 * dis


_k3 = pl.pallas_call(
    _k3_body,
    out_shape=jax.ShapeDtypeStruct((N, D_HID), jnp.float32),
)


def _k4_body(s3_ref, g3_ref, dis_ref, w3_ref, b3_ref, batch_ref, out_ref):
    dis = dis_ref[...]
    agg = (s3_ref[pl.ds(0, N), :] + s3_ref[pl.ds(NP, N), :] + g3_ref[...])
    p3 = dis * agg
    h3 = jnp.maximum(jnp.dot(p3, w3_ref[...],
                             preferred_element_type=jnp.float32)
                     + b3_ref[...], 0.0)
    gids = lax.broadcasted_iota(jnp.int32, (NG, N), 0)
    onehot_t = (gids == batch_ref[...]).astype(jnp.float32)
    sums = jnp.dot(onehot_t, h3, preferred_element_type=jnp.float32)
    cnt = jnp.sum(onehot_t, axis=1, keepdims=True)
    out_ref[...] = sums / jnp.maximum(cnt, 1.0)


_k4 = pl.pallas_call(
    _k4_body,
    out_shape=jax.ShapeDtypeStruct((NG, D_IN), jnp.float32),
)


def kernel(x, edge_index, batch, W1, b1, W2, b2, W3, b3):
    src = edge_index[0].astype(jnp.int32).reshape(NW * NCHUNK, CH)
    dst = edge_index[1].astype(jnp.int32).reshape(NW * NCHUNK, CH)
    batch_r = batch.astype(jnp.int32).reshape(1, N)

    degp = _deg_kernel(dst)
    g1, dis = _k1(x, W1, degp)

    s1 = _prop128(src, dst, g1)
    g2 = _k2(s1, g1, dis, b1.reshape(1, -1), W2)

    s2 = _prop64(src, dst, g2)
    g3 = _k3(s2, g2, dis, b2.reshape(1, -1))

    s3 = _prop64(src, dst, g3)
    out = _k4(s3, g3, dis, W3, b3.reshape(1, -1), batch_r)
    return out


# speedup vs baseline: 1.0917x; 1.0108x over previous
import functools

import jax
import jax.numpy as jnp
from jax import lax
from jax.experimental import pallas as pl
from jax.experimental.pallas import tpu as pltpu
from jax.experimental.pallas import tpu_sc as plsc

N = 10000
E = 320000
D_IN = 128
D_HID = 64
NG = 128

NC = 2
NS = 16
L = 16
NW = NC * NS
EPW = E // NW
CH = 80
NCHUNK = EPW // CH
NPAIR = (NCHUNK - 3) // 2
NP = 10240
ZPT = NP // NS
ZB = 32
DEGW = 16


def _sc_mesh():
    return plsc.VectorSubcoreMesh(
        core_axis_name="c", subcore_axis_name="s",
        num_cores=NC, num_subcores=NS)


def _zero_vmem(ref, rows, d):
    def body(i, _):
        for k in range(d // L):
            ref[i, pl.ds(k * L, L)] = jnp.zeros((L,), jnp.float32)
        return 0
    lax.fori_loop(0, rows, body, 0)


@functools.partial(
    pl.kernel,
    out_type=jax.ShapeDtypeStruct((NC * NP, DEGW), jnp.float32),
    mesh=_sc_mesh(),
    scratch_types=[
        pltpu.VMEM((NCHUNK, CH), jnp.int32),
        pltpu.VMEM((CH, DEGW), jnp.float32),
        pltpu.VMEM((ZB, DEGW), jnp.float32),
        pltpu.VMEM_SHARED((NP, DEGW), jnp.float32),
        pltpu.SemaphoreType.DMA,
        pltpu.SemaphoreType.DMA,
    ],
    name="gcn_deg",
    compiler_params=pltpu.CompilerParams(use_tc_tiling_on_sc=False),
)
def _deg_kernel(dst_hbm, out_hbm, dstall, ones_v, zero_v, acc_sh, semi, sems):
    c = lax.axis_index("c")
    s = lax.axis_index("s")
    wid = s * NC + c

    pltpu.async_copy(dst_hbm.at[pl.ds(wid * NCHUNK, NCHUNK)], dstall, semi)

    def fill_ones(i, _):
        ones_v[i, :] = jnp.ones((L,), jnp.float32)
        return 0
    lax.fori_loop(0, CH, fill_ones, 0)
    _zero_vmem(zero_v, ZB, DEGW)

    def zfire(i, _):
        pltpu.async_copy(
            zero_v,
            acc_sh.at[pl.ds(pl.multiple_of(s * ZPT + i * ZB, 8), ZB)], sems)
        return 0
    lax.fori_loop(0, ZPT // ZB, zfire, 0)

    def zdrain(i, _):
        pltpu.make_async_copy(
            zero_v,
            acc_sh.at[pl.ds(pl.multiple_of(s * ZPT + i * ZB, 8), ZB)],
            sems).wait()
        return 0
    lax.fori_loop(0, ZPT // ZB, zdrain, 0)
    pltpu.make_async_copy(dst_hbm.at[pl.ds(0, NCHUNK)], dstall, semi).wait()
    plsc.subcore_barrier()

    def fire(j, _):
        pltpu.async_copy(ones_v, acc_sh.at[dstall.at[j]], sems, add=True)
        return 0
    lax.fori_loop(0, NCHUNK, fire, 0)

    def drain(j, _):
        pltpu.make_async_copy(ones_v, acc_sh.at[dstall.at[j]], sems).wait()
        return 0
    lax.fori_loop(0, NCHUNK, drain, 0)

    plsc.subcore_barrier()
    row0 = pl.multiple_of(s * ZPT, 8)
    pltpu.sync_copy(acc_sh.at[pl.ds(row0, ZPT)],
                    out_hbm.at[pl.ds(pl.multiple_of(c * NP + s * ZPT, 8), ZPT)])


def _make_prop(d, nbuf, phase_lens):
    ph = max(phase_lens)

    @functools.partial(
        pl.kernel,
        out_type=jax.ShapeDtypeStruct((NC * NP, d), jnp.float32),
        mesh=_sc_mesh(),
        scratch_types=(
            [pltpu.VMEM((ph, CH), jnp.int32),
             pltpu.VMEM((ph, CH), jnp.int32)]
            + [pltpu.VMEM((CH, d), jnp.float32)] * nbuf
            + [pltpu.VMEM((ZB, d), jnp.float32),
               pltpu.VMEM_SHARED((NP, d), jnp.float32),
               pltpu.SemaphoreType.DMA]
            + [pltpu.SemaphoreType.DMA] * nbuf
            + [pltpu.SemaphoreType.DMA] * nbuf
        ),
        name=f"gcn_prop_{d}",
        compiler_params=pltpu.CompilerParams(use_tc_tiling_on_sc=False),
    )
    def prop(src_hbm, dst_hbm, g_hbm, out_hbm, srcph, dstph, *rest):
        rowsb = rest[:nbuf]
        zero_v = rest[nbuf]
        acc_sh = rest[nbuf + 1]
        semi = rest[nbuf + 2]
        semg = rest[nbuf + 3:nbuf + 3 + nbuf]
        sems = rest[nbuf + 3 + nbuf:]

        c = lax.axis_index("c")
        s = lax.axis_index("s")
        wid = s * NC + c

        def issue_idx(row0, ln):
            base = wid * NCHUNK + row0
            pltpu.async_copy(src_hbm.at[pl.ds(base, ln)],
                             srcph.at[pl.ds(0, ln)], semi)
            pltpu.async_copy(dst_hbm.at[pl.ds(base, ln)],
                             dstph.at[pl.ds(0, ln)], semi)

        def wait_idx(ln):
            pltpu.make_async_copy(src_hbm.at[pl.ds(0, ln)],
                                  srcph.at[pl.ds(0, ln)], semi).wait()
            pltpu.make_async_copy(dst_hbm.at[pl.ds(0, ln)],
                                  dstph.at[pl.ds(0, ln)], semi).wait()

        def step(jl, b, ln):
            pltpu.make_async_copy(g_hbm.at[srcph.at[jl]], rowsb[b],
                                  semg[b]).wait()
            pltpu.async_copy(rowsb[b], acc_sh.at[dstph.at[jl]], sems[b],
                             add=True)
            pltpu.make_async_copy(rowsb[b], acc_sh.at[dstph.at[jl]],
                                  sems[b]).wait()

            @pl.when(jl + nbuf < ln)
            def _():
                pltpu.async_copy(g_hbm.at[srcph.at[jl + nbuf]], rowsb[b],
                                 semg[b])

        issue_idx(0, phase_lens[0])
        _zero_vmem(zero_v, ZB, d)

        def zfire(i, _):
            pltpu.async_copy(
                zero_v,
                acc_sh.at[pl.ds(pl.multiple_of(s * ZPT + i * ZB, 8), ZB)],
                sems[0])
            return 0
        lax.fori_loop(0, ZPT // ZB, zfire, 0)

        def zdrain(i, _):
            pltpu.make_async_copy(
                zero_v,
                acc_sh.at[pl.ds(pl.multiple_of(s * ZPT + i * ZB, 8), ZB)],
                sems[0]).wait()
            return 0
        lax.fori_loop(0, ZPT // ZB, zdrain, 0)
        wait_idx(phase_lens[0])
        for b in range(min(nbuf, phase_lens[0])):
            pltpu.async_copy(g_hbm.at[srcph.at[b]], rowsb[b], semg[b])
        plsc.subcore_barrier()

        row0 = 0
        for pi, ln in enumerate(phase_lens):
            if pi > 0:
                issue_idx(row0, ln)
                wait_idx(ln)
                for b in range(min(nbuf, ln)):
                    pltpu.async_copy(g_hbm.at[srcph.at[b]], rowsb[b], semg[b])
            niter = ln // nbuf
            rem = ln % nbuf

            def body(t, _):
                for b in range(nbuf):
                    step(t * nbuf + b, b, ln)
                return 0
            lax.fori_loop(0, niter, body, 0)
            for r in range(rem):
                step(niter * nbuf + r, r, ln)
            row0 += ln

        plsc.subcore_barrier()
        pltpu.sync_copy(acc_sh.at[pl.ds(pl.multiple_of(s * ZPT, 8), ZPT)],
                        out_hbm.at[pl.ds(pl.multiple_of(c * NP + s * ZPT, 8),
                                         ZPT)])

    return prop


_prop128 = _make_prop(D_IN, 3, [63, 62])
_prop64 = _make_prop(D_HID, 4, [NCHUNK])


def _k1_body(x_ref, w1_ref, degp_ref, g1_ref, dis_ref):
    deg = (degp_ref[pl.ds(0, N), pl.ds(0, 1)]
           + degp_ref[pl.ds(NP, N), pl.ds(0, 1)] + 1.0)
    dis = lax.rsqrt(deg)
    dis_ref[...] = dis
    t1 = jnp.dot(x_ref[...], w1_ref[...], preferred_element_type=jnp.float32)
    g1_ref[...] = t1 * dis


_k1 = pl.pallas_call(
    _k1_body,
    out_shape=(jax.ShapeDtypeStruct((N, D_IN), jnp.float32),
               jax.ShapeDtypeStruct((N, 1), jnp.float32)),
)


def _k2_body(s1_ref, g1_ref, dis_ref, b1_ref, w2_ref, g2_ref):
    dis = dis_ref[...]
    agg = (s1_ref[pl.ds(0, N), :] + s1_ref[pl.ds(NP, N), :] + g1_ref[...])
    h1 = jnp.maximum(dis * agg + b1_ref[...], 0.0)
    g2_ref[...] = jnp.dot(h1, w2_ref[...],
                          preferred_element_type=jnp.float32) * dis


_k2 = pl.pallas_call(
    _k2_body,
    out_shape=jax.ShapeDtypeStruct((N, D_HID), jnp.float32),
)


def _k3_body(s2_ref, g2_ref, dis_ref, b2_ref, g3_ref):
    dis = dis_ref[...]
    agg = (s2_ref[pl.ds(0, N), :] + s2_ref[pl.ds(NP, N), :] + g2_ref[...])
    h2 = jnp.maximum(dis * agg + b2_ref[...], 0.0)
    g3_ref[...] = h2 * dis


_k3 = pl.pallas_call(
    _k3_body,
    out_shape=jax.ShapeDtypeStruct((N, D_HID), jnp.float32),
)


def _k4_body(s3_ref, g3_ref, dis_ref, w3_ref, b3_ref, batch_ref, out_ref):
    dis = dis_ref[...]
    agg = (s3_ref[pl.ds(0, N), :] + s3_ref[pl.ds(NP, N), :] + g3_ref[...])
    p3 = dis * agg
    h3 = jnp.maximum(jnp.dot(p3, w3_ref[...],
                             preferred_element_type=jnp.float32)
                     + b3_ref[...], 0.0)
    gids = lax.broadcasted_iota(jnp.int32, (NG, N), 0)
    onehot_t = (gids == batch_ref[...]).astype(jnp.float32)
    sums = jnp.dot(onehot_t, h3, preferred_element_type=jnp.float32)
    cnt = jnp.sum(onehot_t, axis=1, keepdims=True)
    out_ref[...] = sums / jnp.maximum(cnt, 1.0)


_k4 = pl.pallas_call(
    _k4_body,
    out_shape=jax.ShapeDtypeStruct((NG, D_IN), jnp.float32),
)


def kernel(x, edge_index, batch, W1, b1, W2, b2, W3, b3):
    src = edge_index[0].astype(jnp.int32).reshape(NW * NCHUNK, CH)
    dst = edge_index[1].astype(jnp.int32).reshape(NW * NCHUNK, CH)
    batch_r = batch.astype(jnp.int32).reshape(1, N)

    degp = _deg_kernel(dst)
    g1, dis = _k1(x, W1, degp)

    s1 = _prop128(src, dst, g1)
    g2 = _k2(s1, g1, dis, b1.reshape(1, -1), W2)

    s2 = _prop64(src, dst, g2)
    g3 = _k3(s2, g2, dis, b2.reshape(1, -1))

    s3 = _prop64(src, dst, g3)
    out = _k4(s3, g3, dis, W3, b3.reshape(1, -1), batch_r)
    return out
